# baseline (speedup 1.0000x reference)
import jax
import jax.numpy as jnp
from jax import lax
from jax.experimental import pallas as pl
from jax.experimental.pallas import tpu as pltpu
from jax.experimental.pallas import tpu_sc as plsc

N = 10000
NE = 10000
D = 128
E = 320000

NC = 2
NS = 16
NW = NC * NS
PER_W = E // NW
C = 80
NIT = PER_W // C
NEP = 10240
SROWS = NEP // NS
ZR = 128

_mesh = plsc.VectorSubcoreMesh(core_axis_name="c", subcore_axis_name="s")


NG = 80
CH = 128
EP = NW * NG * CH
DUMP = NEP - 1


def _agg_body(tab_hbm, gid_hbm, sid_hbm, out_hbm,
              vid, did, rows, accum, isem, gsem, ssem):
    cid = lax.axis_index("c")
    sid_ = lax.axis_index("s")
    wid = sid_ * NC + cid
    wbase = wid * (NG * CH)

    def fire_ids(g):
        b = pl.multiple_of(wbase + g * CH, 8)
        pltpu.async_copy(gid_hbm.at[pl.ds(b, CH)], vid.at[lax.rem(g, 2)],
                         isem.at[lax.rem(g, 2)])
        pltpu.async_copy(sid_hbm.at[pl.ds(b, CH)], did.at[lax.rem(g, 3)],
                         isem.at[lax.rem(g, 2)])

    def drain_ids(g):
        b = pl.multiple_of(wbase + g * CH, 8)
        pltpu.make_async_copy(gid_hbm.at[pl.ds(b, CH)], vid.at[lax.rem(g, 2)],
                              isem.at[lax.rem(g, 2)]).wait()
        pltpu.make_async_copy(sid_hbm.at[pl.ds(b, CH)], did.at[lax.rem(g, 3)],
                              isem.at[lax.rem(g, 2)]).wait()

    def fire_gather(g, slot):
        pltpu.async_copy(tab_hbm.at[vid.at[slot]], rows.at[slot],
                         gsem.at[slot])

    def drain_gather(g, slot):
        pltpu.make_async_copy(tab_hbm.at[vid.at[slot]], rows.at[slot],
                              gsem.at[slot]).wait()

    def fire_scatter(g, slot):
        pltpu.async_copy(rows.at[slot], accum.at[did.at[lax.rem(g, 3)]],
                         ssem, add=True)

    def drain_scatter(g, slot):
        pltpu.make_async_copy(rows.at[slot], accum.at[did.at[lax.rem(g, 3)]],
                              ssem).wait()

    fire_ids(0)
    fire_ids(1)
    zbuf = rows.at[0]

    def zfill(r, carry):
        def zfill_c(c, _):
            zbuf[r, pl.ds(c * 16, 16)] = jnp.zeros((16,), jnp.float32)
            return _
        return lax.fori_loop(0, D // 16, zfill_c, carry)
    lax.fori_loop(0, CH, zfill, 0)
    for k in range(SROWS // CH):
        pltpu.async_copy(zbuf, accum.at[pl.ds(sid_ * SROWS + k * CH, CH)],
                         ssem)
    for k in range(SROWS // CH):
        pltpu.make_async_copy(zbuf, accum.at[pl.ds(sid_ * SROWS + k * CH, CH)],
                              ssem).wait()
    drain_ids(0)
    plsc.subcore_barrier()
    fire_gather(0, 0)

    def piter(g, carry):
        slot = lax.rem(g, 2)

        @pl.when(g > 0)
        def _():
            drain_scatter(g - 1, 1 - slot)

        @pl.when(g + 1 < NG)
        def _():
            drain_ids(g + 1)
            fire_gather(g + 1, 1 - slot)

        drain_gather(g, slot)
        fire_scatter(g, slot)

        @pl.when(g + 2 < NG)
        def _():
            fire_ids(g + 2)
        return carry
    lax.fori_loop(0, NG, piter, 0)
    drain_scatter(NG - 1, lax.rem(NG - 1, 2))

    plsc.subcore_barrier()
    r0 = sid_ * SROWS
    pltpu.sync_copy(accum.at[pl.ds(r0, SROWS)],
                    out_hbm.at[pl.ds(cid * NEP + r0, SROWS)])


_agg = pl.kernel(
    _agg_body,
    out_type=jax.ShapeDtypeStruct((NC * NEP, D), jnp.float32),
    scratch_types=[
        pltpu.VMEM((2, CH), jnp.int32),
        pltpu.VMEM((3, CH), jnp.int32),
        pltpu.VMEM((2, CH, D), jnp.float32),
        pltpu.VMEM_SHARED((NEP, D), jnp.float32),
        pltpu.SemaphoreType.DMA((2,)),
        pltpu.SemaphoreType.DMA((2,)),
        pltpu.SemaphoreType.DMA,
    ],
    mesh=_mesh,
)


CW = 16
CWORDS = NEP * CW
CTILE = CWORDS // NS
CZ = 2048


def _counts_body(ids_hbm, out_hbm, dbuf, sbuf, ones, zbuf, accum):
    cid = lax.axis_index("c")
    sid = lax.axis_index("s")
    wid = sid * NC + cid

    def ofill(r, carry):
        ones[pl.ds(r * 16, 16)] = jnp.ones((16,), jnp.float32)
        return carry
    lax.fori_loop(0, C // 16, ofill, 0)

    def zfill(r, carry):
        zbuf[pl.ds(r * 16, 16)] = jnp.zeros((16,), jnp.float32)
        return carry
    lax.fori_loop(0, CZ // 16, zfill, 0)
    for k in range(CTILE // CZ):
        pltpu.sync_copy(zbuf, accum.at[pl.ds(sid * CTILE + k * CZ, CZ)])
    plsc.subcore_barrier()

    def body(i, carry):
        base = pl.multiple_of(wid * PER_W + i * C, 8)
        pltpu.sync_copy(ids_hbm.at[pl.ds(base, C)], dbuf.at[0])

        def scale(j, carry2):
            sbuf[0, pl.ds(j * 16, 16)] = dbuf[0, pl.ds(j * 16, 16)] * CW
            return carry2
        lax.fori_loop(0, C // 16, scale, 0)
        pltpu.sync_copy(ones, accum.at[sbuf.at[0]], add=True)
        return carry
    lax.fori_loop(0, NIT, body, 0)

    plsc.subcore_barrier()
    r0 = sid * CTILE
    pltpu.sync_copy(accum.at[pl.ds(r0, CTILE)],
                    out_hbm.at[pl.ds(cid * CWORDS + r0, CTILE)])


_counts = pl.kernel(
    _counts_body,
    out_type=jax.ShapeDtypeStruct((NC * CWORDS,), jnp.float32),
    scratch_types=[
        pltpu.VMEM((1, C), jnp.int32),
        pltpu.VMEM((1, C), jnp.int32),
        pltpu.VMEM((C,), jnp.float32),
        pltpu.VMEM((CZ,), jnp.float32),
        pltpu.VMEM_SHARED((CWORDS,), jnp.float32),
    ],
    mesh=_mesh,
)


BR = 1000


def _mm_body(x_ref, w_ref, o_ref):
    o_ref[...] = lax.dot_general(
        x_ref[...], w_ref[...], (((1,), (1,)), ((), ())),
        preferred_element_type=jnp.float32)


def _mm(x, w):
    return pl.pallas_call(
        _mm_body,
        grid=(N // BR,),
        in_specs=[
            pl.BlockSpec((BR, D), lambda i: (i, 0)),
            pl.BlockSpec((D, D), lambda i: (0, 0)),
        ],
        out_specs=pl.BlockSpec((BR, D), lambda i: (i, 0)),
        out_shape=jax.ShapeDtypeStruct((N, D), jnp.float32),
    )(x, w)


def _recip_body(cp_ref, o_ref):
    a = cp_ref[...]
    cnt = a[:NE, :1] + a[NEP:NEP + NE, :1]
    o_ref[...] = 1.0 / jnp.maximum(cnt, 1.0)


def _recip_counts(cp):
    return pl.pallas_call(
        _recip_body,
        grid=(1,),
        in_specs=[pl.BlockSpec((NC * NEP, CW), lambda i: (0, 0))],
        out_specs=pl.BlockSpec((NE, 1), lambda i: (0, 0)),
        out_shape=jax.ShapeDtypeStruct((NE, 1), jnp.float32),
    )(cp)


def _ec_body(p0_ref, p1_ref, rec_ref, o_ref):
    o_ref[...] = (p0_ref[0] + p1_ref[0]) * rec_ref[...]


def _edge_combine(ep, rec):
    ep3 = ep.reshape(NC, NEP, D)
    return pl.pallas_call(
        _ec_body,
        grid=(NE // BR,),
        in_specs=[
            pl.BlockSpec((1, BR, D), lambda i: (0, i, 0)),
            pl.BlockSpec((1, BR, D), lambda i: (1, i, 0)),
            pl.BlockSpec((BR, 1), lambda i: (i, 0)),
        ],
        out_specs=pl.BlockSpec((BR, D), lambda i: (i, 0)),
        out_shape=jax.ShapeDtypeStruct((NE, D), jnp.float32),
    )(ep3, ep3, rec)


def _vc1_body(eps_ref, y_ref, v0_ref, v1_ref, w_ref, o_ref):
    x = (1.0 + eps_ref[0]) * y_ref[...] + v0_ref[0] + v1_ref[0]
    x = jnp.maximum(x, 0.0)
    o_ref[...] = lax.dot_general(
        x, w_ref[...], (((1,), (1,)), ((), ())),
        preferred_element_type=jnp.float32)


def _vert_combine_relu_mm(y, vp, w2, eps):
    vp3 = vp.reshape(NC, NEP, D)
    return pl.pallas_call(
        _vc1_body,
        grid=(N // BR,),
        in_specs=[
            pl.BlockSpec(memory_space=pltpu.SMEM),
            pl.BlockSpec((BR, D), lambda i: (i, 0)),
            pl.BlockSpec((1, BR, D), lambda i: (0, i, 0)),
            pl.BlockSpec((1, BR, D), lambda i: (1, i, 0)),
            pl.BlockSpec((D, D), lambda i: (0, 0)),
        ],
        out_specs=pl.BlockSpec((BR, D), lambda i: (i, 0)),
        out_shape=jax.ShapeDtypeStruct((N, D), jnp.float32),
    )(eps, y, vp3, vp3, w2)


def _vc2_body(eps_ref, y_ref, v0_ref, v1_ref, o_ref):
    o_ref[...] = (1.0 + eps_ref[0]) * y_ref[...] + v0_ref[0] + v1_ref[0]


def _vert_combine(y, vp, eps):
    vp3 = vp.reshape(NC, NEP, D)
    return pl.pallas_call(
        _vc2_body,
        grid=(N // BR,),
        in_specs=[
            pl.BlockSpec(memory_space=pltpu.SMEM),
            pl.BlockSpec((BR, D), lambda i: (i, 0)),
            pl.BlockSpec((1, BR, D), lambda i: (0, i, 0)),
            pl.BlockSpec((1, BR, D), lambda i: (1, i, 0)),
        ],
        out_specs=pl.BlockSpec((BR, D), lambda i: (i, 0)),
        out_shape=jax.ShapeDtypeStruct((N, D), jnp.float32),
    )(eps, y, vp3, vp3)


def kernel(x, hyperedge_index, W1, eps1, W2, eps2):
    vertex = hyperedge_index[0]
    edges = hyperedge_index[1]
    pad = EP - E
    vertex_g = jnp.pad(vertex, (0, pad))
    edges_g = jnp.pad(edges, (0, pad))
    vertex_s = jnp.pad(vertex, (0, pad), constant_values=DUMP)
    edges_s = jnp.pad(edges, (0, pad), constant_values=DUMP)

    cp = _counts(edges).reshape(NC * NEP, CW)
    rec = _recip_counts(cp)

    y1 = _mm(x, W1)
    ep1 = _agg(y1, vertex_g, edges_s)
    xe1 = _edge_combine(ep1, rec)
    vp1 = _agg(xe1, edges_g, vertex_s)
    y2 = _vert_combine_relu_mm(y1, vp1, W2, eps1)

    ep2 = _agg(y2, vertex_g, edges_s)
    xe2 = _edge_combine(ep2, rec)
    vp2 = _agg(xe2, edges_g, vertex_s)
    x2 = _vert_combine(y2, vp2, eps2)

    return (x2, xe2)


# speedup vs baseline: 3.6522x; 3.6522x over previous
import jax
import jax.numpy as jnp
from jax import lax
from jax.experimental import pallas as pl
from jax.experimental.pallas import tpu as pltpu
from jax.experimental.pallas import tpu_sc as plsc

N = 10000
NE = 10000
D = 128
E = 320000

NC = 2
NS = 16
NW = NC * NS
PER_W = E // NW
C = 80
NIT = PER_W // C
NEP = 10240
SROWS = NEP // NS
ZR = 128

_mesh = plsc.VectorSubcoreMesh(core_axis_name="c", subcore_axis_name="s")


CH = 128
NCHUNK = E // CH
NGBASE = NCHUNK // NW
NGREM = NCHUNK % NW


def _agg_body(tab_hbm, gid_hbm, sid_hbm, out_hbm,
              vid, did, rows, accum, isem, gsem, ssem):
    cid = lax.axis_index("c")
    sid_ = lax.axis_index("s")
    wid = sid_ * NC + cid
    ng = NGBASE + jnp.where(wid < NGREM, 1, 0)
    wbase = (wid * NGBASE + jnp.minimum(wid, NGREM)) * CH

    def fire_ids(g):
        b = pl.multiple_of(wbase + g * CH, 8)
        pltpu.async_copy(gid_hbm.at[pl.ds(b, CH)], vid.at[lax.rem(g, 2)],
                         isem.at[lax.rem(g, 2)])
        pltpu.async_copy(sid_hbm.at[pl.ds(b, CH)], did.at[lax.rem(g, 3)],
                         isem.at[lax.rem(g, 2)])

    def drain_ids(g):
        b = pl.multiple_of(wbase + g * CH, 8)
        pltpu.make_async_copy(gid_hbm.at[pl.ds(b, CH)], vid.at[lax.rem(g, 2)],
                              isem.at[lax.rem(g, 2)]).wait()
        pltpu.make_async_copy(sid_hbm.at[pl.ds(b, CH)], did.at[lax.rem(g, 3)],
                              isem.at[lax.rem(g, 2)]).wait()

    def fire_gather(g, slot):
        pltpu.async_copy(tab_hbm.at[vid.at[slot]], rows.at[slot],
                         gsem.at[slot])

    def drain_gather(g, slot):
        pltpu.make_async_copy(tab_hbm.at[vid.at[slot]], rows.at[slot],
                              gsem.at[slot]).wait()

    def fire_scatter(g, slot):
        pltpu.async_copy(rows.at[slot], accum.at[did.at[lax.rem(g, 3)]],
                         ssem, add=True)

    def drain_scatter(g, slot):
        pltpu.make_async_copy(rows.at[slot], accum.at[did.at[lax.rem(g, 3)]],
                              ssem).wait()

    fire_ids(0)
    fire_ids(1)
    zbuf = rows.at[0]

    def zfill(r, carry):
        def zfill_c(c, _):
            zbuf[r, pl.ds(c * 16, 16)] = jnp.zeros((16,), jnp.float32)
            return _
        return lax.fori_loop(0, D // 16, zfill_c, carry)
    lax.fori_loop(0, CH, zfill, 0)
    for k in range(SROWS // CH):
        pltpu.async_copy(zbuf, accum.at[pl.ds(sid_ * SROWS + k * CH, CH)],
                         ssem)
    for k in range(SROWS // CH):
        pltpu.make_async_copy(zbuf, accum.at[pl.ds(sid_ * SROWS + k * CH, CH)],
                              ssem).wait()
    drain_ids(0)
    plsc.subcore_barrier()
    fire_gather(0, 0)

    def piter(g, carry):
        slot = lax.rem(g, 2)

        @pl.when(g > 0)
        def _():
            drain_scatter(g - 1, 1 - slot)

        @pl.when(g + 1 < ng)
        def _():
            drain_ids(g + 1)
            fire_gather(g + 1, 1 - slot)

        drain_gather(g, slot)
        fire_scatter(g, slot)

        @pl.when(g + 2 < ng)
        def _():
            fire_ids(g + 2)
        return carry
    lax.fori_loop(0, ng, piter, 0)
    drain_scatter(ng - 1, lax.rem(ng - 1, 2))

    plsc.subcore_barrier()
    r0 = sid_ * SROWS
    pltpu.sync_copy(accum.at[pl.ds(r0, SROWS)],
                    out_hbm.at[pl.ds(cid * NEP + r0, SROWS)])


_agg = pl.kernel(
    _agg_body,
    out_type=jax.ShapeDtypeStruct((NC * NEP, D), jnp.float32),
    scratch_types=[
        pltpu.VMEM((2, CH), jnp.int32),
        pltpu.VMEM((3, CH), jnp.int32),
        pltpu.VMEM((2, CH, D), jnp.float32),
        pltpu.VMEM_SHARED((NEP, D), jnp.float32),
        pltpu.SemaphoreType.DMA((2,)),
        pltpu.SemaphoreType.DMA((2,)),
        pltpu.SemaphoreType.DMA,
    ],
    mesh=_mesh,
)


CW = 16
CWORDS = NEP * CW
CTILE = CWORDS // NS
CZ = 2048


def _counts_body(ids_hbm, out_hbm, dbuf, sbuf, ones, zbuf, accum):
    cid = lax.axis_index("c")
    sid = lax.axis_index("s")
    wid = sid * NC + cid

    def ofill(r, carry):
        ones[pl.ds(r * 16, 16)] = jnp.ones((16,), jnp.float32)
        return carry
    lax.fori_loop(0, C // 16, ofill, 0)

    def zfill(r, carry):
        zbuf[pl.ds(r * 16, 16)] = jnp.zeros((16,), jnp.float32)
        return carry
    lax.fori_loop(0, CZ // 16, zfill, 0)
    for k in range(CTILE // CZ):
        pltpu.sync_copy(zbuf, accum.at[pl.ds(sid * CTILE + k * CZ, CZ)])
    plsc.subcore_barrier()

    def body(i, carry):
        base = pl.multiple_of(wid * PER_W + i * C, 8)
        pltpu.sync_copy(ids_hbm.at[pl.ds(base, C)], dbuf.at[0])

        def scale(j, carry2):
            sbuf[0, pl.ds(j * 16, 16)] = dbuf[0, pl.ds(j * 16, 16)] * CW
            return carry2
        lax.fori_loop(0, C // 16, scale, 0)
        pltpu.sync_copy(ones, accum.at[sbuf.at[0]], add=True)
        return carry
    lax.fori_loop(0, NIT, body, 0)

    plsc.subcore_barrier()
    r0 = sid * CTILE
    pltpu.sync_copy(accum.at[pl.ds(r0, CTILE)],
                    out_hbm.at[pl.ds(cid * CWORDS + r0, CTILE)])


_counts = pl.kernel(
    _counts_body,
    out_type=jax.ShapeDtypeStruct((NC * CWORDS,), jnp.float32),
    scratch_types=[
        pltpu.VMEM((1, C), jnp.int32),
        pltpu.VMEM((1, C), jnp.int32),
        pltpu.VMEM((C,), jnp.float32),
        pltpu.VMEM((CZ,), jnp.float32),
        pltpu.VMEM_SHARED((CWORDS,), jnp.float32),
    ],
    mesh=_mesh,
)


BR = 1000


def _mm_body(x_ref, w_ref, o_ref):
    o_ref[...] = lax.dot_general(
        x_ref[...], w_ref[...], (((1,), (1,)), ((), ())),
        preferred_element_type=jnp.float32)


def _mm(x, w):
    return pl.pallas_call(
        _mm_body,
        grid=(N // BR,),
        in_specs=[
            pl.BlockSpec((BR, D), lambda i: (i, 0)),
            pl.BlockSpec((D, D), lambda i: (0, 0)),
        ],
        out_specs=pl.BlockSpec((BR, D), lambda i: (i, 0)),
        out_shape=jax.ShapeDtypeStruct((N, D), jnp.float32),
    )(x, w)


def _recip_body(cp_ref, o_ref):
    a = cp_ref[...]
    cnt = a[:NE, :1] + a[NEP:NEP + NE, :1]
    o_ref[...] = 1.0 / jnp.maximum(cnt, 1.0)


def _recip_counts(cp):
    return pl.pallas_call(
        _recip_body,
        grid=(1,),
        in_specs=[pl.BlockSpec((NC * NEP, CW), lambda i: (0, 0))],
        out_specs=pl.BlockSpec((NE, 1), lambda i: (0, 0)),
        out_shape=jax.ShapeDtypeStruct((NE, 1), jnp.float32),
    )(cp)


def _ec_body(p0_ref, p1_ref, rec_ref, o_ref):
    o_ref[...] = (p0_ref[0] + p1_ref[0]) * rec_ref[...]


def _edge_combine(ep, rec):
    ep3 = ep.reshape(NC, NEP, D)
    return pl.pallas_call(
        _ec_body,
        grid=(NE // BR,),
        in_specs=[
            pl.BlockSpec((1, BR, D), lambda i: (0, i, 0)),
            pl.BlockSpec((1, BR, D), lambda i: (1, i, 0)),
            pl.BlockSpec((BR, 1), lambda i: (i, 0)),
        ],
        out_specs=pl.BlockSpec((BR, D), lambda i: (i, 0)),
        out_shape=jax.ShapeDtypeStruct((NE, D), jnp.float32),
    )(ep3, ep3, rec)


def _vc1_body(eps_ref, y_ref, v0_ref, v1_ref, w_ref, o_ref):
    x = (1.0 + eps_ref[0]) * y_ref[...] + v0_ref[0] + v1_ref[0]
    x = jnp.maximum(x, 0.0)
    o_ref[...] = lax.dot_general(
        x, w_ref[...], (((1,), (1,)), ((), ())),
        preferred_element_type=jnp.float32)


def _vert_combine_relu_mm(y, vp, w2, eps):
    vp3 = vp.reshape(NC, NEP, D)
    return pl.pallas_call(
        _vc1_body,
        grid=(N // BR,),
        in_specs=[
            pl.BlockSpec(memory_space=pltpu.SMEM),
            pl.BlockSpec((BR, D), lambda i: (i, 0)),
            pl.BlockSpec((1, BR, D), lambda i: (0, i, 0)),
            pl.BlockSpec((1, BR, D), lambda i: (1, i, 0)),
            pl.BlockSpec((D, D), lambda i: (0, 0)),
        ],
        out_specs=pl.BlockSpec((BR, D), lambda i: (i, 0)),
        out_shape=jax.ShapeDtypeStruct((N, D), jnp.float32),
    )(eps, y, vp3, vp3, w2)


def _vc2_body(eps_ref, y_ref, v0_ref, v1_ref, o_ref):
    o_ref[...] = (1.0 + eps_ref[0]) * y_ref[...] + v0_ref[0] + v1_ref[0]


def _vert_combine(y, vp, eps):
    vp3 = vp.reshape(NC, NEP, D)
    return pl.pallas_call(
        _vc2_body,
        grid=(N // BR,),
        in_specs=[
            pl.BlockSpec(memory_space=pltpu.SMEM),
            pl.BlockSpec((BR, D), lambda i: (i, 0)),
            pl.BlockSpec((1, BR, D), lambda i: (0, i, 0)),
            pl.BlockSpec((1, BR, D), lambda i: (1, i, 0)),
        ],
        out_specs=pl.BlockSpec((BR, D), lambda i: (i, 0)),
        out_shape=jax.ShapeDtypeStruct((N, D), jnp.float32),
    )(eps, y, vp3, vp3)


def kernel(x, hyperedge_index, W1, eps1, W2, eps2):
    vertex = hyperedge_index[0]
    edges = hyperedge_index[1]

    cp = _counts(edges).reshape(NC * NEP, CW)
    rec = _recip_counts(cp)

    y1 = _mm(x, W1)
    ep1 = _agg(y1, vertex, edges)
    xe1 = _edge_combine(ep1, rec)
    vp1 = _agg(xe1, edges, vertex)
    y2 = _vert_combine_relu_mm(y1, vp1, W2, eps1)

    ep2 = _agg(y2, vertex, edges)
    xe2 = _edge_combine(ep2, rec)
    vp2 = _agg(xe2, edges, vertex)
    x2 = _vert_combine(y2, vp2, eps2)

    return (x2, xe2)


# speedup vs baseline: 3.9769x; 1.0889x over previous
import jax
import jax.numpy as jnp
from jax import lax
from jax.experimental import pallas as pl
from jax.experimental.pallas import tpu as pltpu
from jax.experimental.pallas import tpu_sc as plsc

N = 10000
NE = 10000
D = 128
E = 320000

NC = 2
NS = 16
NW = NC * NS
PER_W = E // NW
C = 80
NIT = PER_W // C
NEP = 10240
SROWS = NEP // NS
ZR = 128

_mesh = plsc.VectorSubcoreMesh(core_axis_name="c", subcore_axis_name="s")


CH = 128
NCHUNK = E // CH
NGBASE = NCHUNK // NW
NGREM = NCHUNK % NW


def _agg_body(tab_hbm, gid_hbm, sid_hbm, out_hbm,
              vid, did, rows, accum, isem, gsem, ssem):
    cid = lax.axis_index("c")
    sid_ = lax.axis_index("s")
    wid = sid_ * NC + cid
    ng = NGBASE + jnp.where(wid < NGREM, 1, 0)
    wbase = (wid * NGBASE + jnp.minimum(wid, NGREM)) * CH

    def fire_ids(g):
        b = pl.multiple_of(wbase + g * CH, 8)
        pltpu.async_copy(gid_hbm.at[pl.ds(b, CH)], vid.at[lax.rem(g, 2)],
                         isem.at[lax.rem(g, 2)])
        pltpu.async_copy(sid_hbm.at[pl.ds(b, CH)], did.at[lax.rem(g, 3)],
                         isem.at[lax.rem(g, 2)])

    def drain_ids(g):
        b = pl.multiple_of(wbase + g * CH, 8)
        pltpu.make_async_copy(gid_hbm.at[pl.ds(b, CH)], vid.at[lax.rem(g, 2)],
                              isem.at[lax.rem(g, 2)]).wait()
        pltpu.make_async_copy(sid_hbm.at[pl.ds(b, CH)], did.at[lax.rem(g, 3)],
                              isem.at[lax.rem(g, 2)]).wait()

    def fire_gather(g, slot):
        pltpu.async_copy(tab_hbm.at[vid.at[slot]], rows.at[slot],
                         gsem.at[slot])

    def drain_gather(g, slot):
        pltpu.make_async_copy(tab_hbm.at[vid.at[slot]], rows.at[slot],
                              gsem.at[slot]).wait()

    def fire_scatter(g, slot):
        pltpu.async_copy(rows.at[slot], accum.at[did.at[lax.rem(g, 3)]],
                         ssem, add=True)

    def drain_scatter(g, slot):
        pltpu.make_async_copy(rows.at[slot], accum.at[did.at[lax.rem(g, 3)]],
                              ssem).wait()

    fire_ids(0)
    fire_ids(1)
    zbuf = rows.at[0]

    def zfill(r, carry):
        def zfill_c(c, _):
            zbuf[r, pl.ds(c * 16, 16)] = jnp.zeros((16,), jnp.float32)
            return _
        return lax.fori_loop(0, D // 16, zfill_c, carry)
    lax.fori_loop(0, CH, zfill, 0)
    for k in range(SROWS // CH):
        pltpu.async_copy(zbuf, accum.at[pl.ds(sid_ * SROWS + k * CH, CH)],
                         ssem)
    for k in range(SROWS // CH):
        pltpu.make_async_copy(zbuf, accum.at[pl.ds(sid_ * SROWS + k * CH, CH)],
                              ssem).wait()
    drain_ids(0)
    plsc.subcore_barrier()
    fire_gather(0, 0)

    def piter(g, carry):
        slot = lax.rem(g, 2)

        @pl.when(g > 0)
        def _():
            drain_scatter(g - 1, 1 - slot)

        @pl.when(g + 1 < ng)
        def _():
            drain_ids(g + 1)
            fire_gather(g + 1, 1 - slot)

        drain_gather(g, slot)
        fire_scatter(g, slot)

        @pl.when(g + 2 < ng)
        def _():
            fire_ids(g + 2)
        return carry
    lax.fori_loop(0, ng, piter, 0)
    drain_scatter(ng - 1, lax.rem(ng - 1, 2))

    plsc.subcore_barrier()
    r0 = sid_ * SROWS
    pltpu.sync_copy(accum.at[pl.ds(r0, SROWS)],
                    out_hbm.at[pl.ds(cid * NEP + r0, SROWS)])


_agg = pl.kernel(
    _agg_body,
    out_type=jax.ShapeDtypeStruct((NC * NEP, D), jnp.float32),
    scratch_types=[
        pltpu.VMEM((2, CH), jnp.int32),
        pltpu.VMEM((3, CH), jnp.int32),
        pltpu.VMEM((2, CH, D), jnp.float32),
        pltpu.VMEM_SHARED((NEP, D), jnp.float32),
        pltpu.SemaphoreType.DMA((2,)),
        pltpu.SemaphoreType.DMA((2,)),
        pltpu.SemaphoreType.DMA,
    ],
    mesh=_mesh,
)


CW = 16
CWORDS = NEP * CW
CTILE = CWORDS // NS
CZ = 2048


def _counts_body(ids_hbm, out_hbm, dbuf, sbuf, ones, zbuf, accum, isem, ssem):
    cid = lax.axis_index("c")
    sid = lax.axis_index("s")
    wid = sid * NC + cid
    ng = NGBASE + jnp.where(wid < NGREM, 1, 0)
    wbase = (wid * NGBASE + jnp.minimum(wid, NGREM)) * CH

    def fire_ids(g):
        b = pl.multiple_of(wbase + g * CH, 8)
        pltpu.async_copy(ids_hbm.at[pl.ds(b, CH)], dbuf.at[lax.rem(g, 2)],
                         isem.at[lax.rem(g, 2)])

    def drain_ids(g):
        b = pl.multiple_of(wbase + g * CH, 8)
        pltpu.make_async_copy(ids_hbm.at[pl.ds(b, CH)],
                              dbuf.at[lax.rem(g, 2)],
                              isem.at[lax.rem(g, 2)]).wait()

    def fire_scatter(slot):
        pltpu.async_copy(ones, accum.at[sbuf.at[slot]], ssem, add=True)

    def drain_scatter(slot):
        pltpu.make_async_copy(ones, accum.at[sbuf.at[slot]], ssem).wait()

    fire_ids(0)
    fire_ids(1)

    def ofill(r, carry):
        ones[pl.ds(r * 16, 16)] = jnp.ones((16,), jnp.float32)
        return carry
    lax.fori_loop(0, CH // 16, ofill, 0)

    def zfill(r, carry):
        zbuf[pl.ds(r * 16, 16)] = jnp.zeros((16,), jnp.float32)
        return carry
    lax.fori_loop(0, CZ // 16, zfill, 0)
    for k in range(CTILE // CZ):
        pltpu.async_copy(zbuf, accum.at[pl.ds(sid * CTILE + k * CZ, CZ)],
                         ssem)
    for k in range(CTILE // CZ):
        pltpu.make_async_copy(zbuf, accum.at[pl.ds(sid * CTILE + k * CZ, CZ)],
                              ssem).wait()
    plsc.subcore_barrier()

    def body(g, carry):
        slot = lax.rem(g, 2)

        @pl.when(g > 0)
        def _():
            drain_scatter(1 - slot)

        drain_ids(g)

        def scale(j, carry2):
            sbuf[slot, pl.ds(j * 16, 16)] = dbuf[slot, pl.ds(j * 16, 16)] * CW
            return carry2
        lax.fori_loop(0, CH // 16, scale, 0)
        fire_scatter(slot)

        @pl.when(g + 2 < ng)
        def _():
            fire_ids(g + 2)
        return carry
    lax.fori_loop(0, ng, body, 0)
    drain_scatter(lax.rem(ng - 1, 2))

    plsc.subcore_barrier()
    r0 = sid * CTILE
    pltpu.sync_copy(accum.at[pl.ds(r0, CTILE)],
                    out_hbm.at[pl.ds(cid * CWORDS + r0, CTILE)])


_counts = pl.kernel(
    _counts_body,
    out_type=jax.ShapeDtypeStruct((NC * CWORDS,), jnp.float32),
    scratch_types=[
        pltpu.VMEM((2, CH), jnp.int32),
        pltpu.VMEM((2, CH), jnp.int32),
        pltpu.VMEM((CH,), jnp.float32),
        pltpu.VMEM((CZ,), jnp.float32),
        pltpu.VMEM_SHARED((CWORDS,), jnp.float32),
        pltpu.SemaphoreType.DMA((2,)),
        pltpu.SemaphoreType.DMA,
    ],
    mesh=_mesh,
)


BR = 1000


def _mm_body(x_ref, w_ref, o_ref):
    o_ref[...] = lax.dot_general(
        x_ref[...], w_ref[...], (((1,), (1,)), ((), ())),
        preferred_element_type=jnp.float32)


def _mm(x, w):
    return pl.pallas_call(
        _mm_body,
        grid=(N // BR,),
        in_specs=[
            pl.BlockSpec((BR, D), lambda i: (i, 0)),
            pl.BlockSpec((D, D), lambda i: (0, 0)),
        ],
        out_specs=pl.BlockSpec((BR, D), lambda i: (i, 0)),
        out_shape=jax.ShapeDtypeStruct((N, D), jnp.float32),
    )(x, w)


def _recip_body(cp_ref, o_ref):
    a = cp_ref[...]
    cnt = a[:NE, :1] + a[NEP:NEP + NE, :1]
    o_ref[...] = 1.0 / jnp.maximum(cnt, 1.0)


def _recip_counts(cp):
    return pl.pallas_call(
        _recip_body,
        grid=(1,),
        in_specs=[pl.BlockSpec((NC * NEP, CW), lambda i: (0, 0))],
        out_specs=pl.BlockSpec((NE, 1), lambda i: (0, 0)),
        out_shape=jax.ShapeDtypeStruct((NE, 1), jnp.float32),
    )(cp)


def _ec_body(p0_ref, p1_ref, rec_ref, o_ref):
    o_ref[...] = (p0_ref[0] + p1_ref[0]) * rec_ref[...]


def _edge_combine(ep, rec):
    ep3 = ep.reshape(NC, NEP, D)
    return pl.pallas_call(
        _ec_body,
        grid=(NE // BR,),
        in_specs=[
            pl.BlockSpec((1, BR, D), lambda i: (0, i, 0)),
            pl.BlockSpec((1, BR, D), lambda i: (1, i, 0)),
            pl.BlockSpec((BR, 1), lambda i: (i, 0)),
        ],
        out_specs=pl.BlockSpec((BR, D), lambda i: (i, 0)),
        out_shape=jax.ShapeDtypeStruct((NE, D), jnp.float32),
    )(ep3, ep3, rec)


def _vc1_body(eps_ref, y_ref, v0_ref, v1_ref, w_ref, o_ref):
    x = (1.0 + eps_ref[0]) * y_ref[...] + v0_ref[0] + v1_ref[0]
    x = jnp.maximum(x, 0.0)
    o_ref[...] = lax.dot_general(
        x, w_ref[...], (((1,), (1,)), ((), ())),
        preferred_element_type=jnp.float32)


def _vert_combine_relu_mm(y, vp, w2, eps):
    vp3 = vp.reshape(NC, NEP, D)
    return pl.pallas_call(
        _vc1_body,
        grid=(N // BR,),
        in_specs=[
            pl.BlockSpec(memory_space=pltpu.SMEM),
            pl.BlockSpec((BR, D), lambda i: (i, 0)),
            pl.BlockSpec((1, BR, D), lambda i: (0, i, 0)),
            pl.BlockSpec((1, BR, D), lambda i: (1, i, 0)),
            pl.BlockSpec((D, D), lambda i: (0, 0)),
        ],
        out_specs=pl.BlockSpec((BR, D), lambda i: (i, 0)),
        out_shape=jax.ShapeDtypeStruct((N, D), jnp.float32),
    )(eps, y, vp3, vp3, w2)


def _vc2_body(eps_ref, y_ref, v0_ref, v1_ref, o_ref):
    o_ref[...] = (1.0 + eps_ref[0]) * y_ref[...] + v0_ref[0] + v1_ref[0]


def _vert_combine(y, vp, eps):
    vp3 = vp.reshape(NC, NEP, D)
    return pl.pallas_call(
        _vc2_body,
        grid=(N // BR,),
        in_specs=[
            pl.BlockSpec(memory_space=pltpu.SMEM),
            pl.BlockSpec((BR, D), lambda i: (i, 0)),
            pl.BlockSpec((1, BR, D), lambda i: (0, i, 0)),
            pl.BlockSpec((1, BR, D), lambda i: (1, i, 0)),
        ],
        out_specs=pl.BlockSpec((BR, D), lambda i: (i, 0)),
        out_shape=jax.ShapeDtypeStruct((N, D), jnp.float32),
    )(eps, y, vp3, vp3)


def kernel(x, hyperedge_index, W1, eps1, W2, eps2):
    vertex = hyperedge_index[0]
    edges = hyperedge_index[1]

    cp = _counts(edges).reshape(NC * NEP, CW)
    rec = _recip_counts(cp)

    y1 = _mm(x, W1)
    ep1 = _agg(y1, vertex, edges)
    xe1 = _edge_combine(ep1, rec)
    vp1 = _agg(xe1, edges, vertex)
    y2 = _vert_combine_relu_mm(y1, vp1, W2, eps1)

    ep2 = _agg(y2, vertex, edges)
    xe2 = _edge_combine(ep2, rec)
    vp2 = _agg(xe2, edges, vertex)
    x2 = _vert_combine(y2, vp2, eps2)

    return (x2, xe2)


# speedup vs baseline: 4.0218x; 1.0113x over previous
import jax
import jax.numpy as jnp
from jax import lax
from jax.experimental import pallas as pl
from jax.experimental.pallas import tpu as pltpu
from jax.experimental.pallas import tpu_sc as plsc

N = 10000
NE = 10000
D = 128
E = 320000

NC = 2
NS = 16
NW = NC * NS
PER_W = E // NW
C = 80
NIT = PER_W // C
NEP = 10240
SROWS = NEP // NS
ZR = 128

_mesh = plsc.VectorSubcoreMesh(core_axis_name="c", subcore_axis_name="s")


CH = 128
NCHUNK = E // CH
NGBASE = NCHUNK // NW
NGREM = NCHUNK % NW


def _agg_body(tab_hbm, gid_hbm, sid_hbm, out_hbm,
              vid, did, rows, accum, isem, gsem, ssem):
    cid = lax.axis_index("c")
    sid_ = lax.axis_index("s")
    wid = sid_ * NC + cid
    ng = NGBASE + jnp.where(wid < NGREM, 1, 0)
    wbase = (wid * NGBASE + jnp.minimum(wid, NGREM)) * CH

    def fire_ids(g):
        b = pl.multiple_of(wbase + g * CH, 8)
        pltpu.async_copy(gid_hbm.at[pl.ds(b, CH)], vid.at[lax.rem(g, 2)],
                         isem.at[lax.rem(g, 2)])
        pltpu.async_copy(sid_hbm.at[pl.ds(b, CH)], did.at[lax.rem(g, 3)],
                         isem.at[lax.rem(g, 2)])

    def drain_ids(g):
        b = pl.multiple_of(wbase + g * CH, 8)
        pltpu.make_async_copy(gid_hbm.at[pl.ds(b, CH)], vid.at[lax.rem(g, 2)],
                              isem.at[lax.rem(g, 2)]).wait()
        pltpu.make_async_copy(sid_hbm.at[pl.ds(b, CH)], did.at[lax.rem(g, 3)],
                              isem.at[lax.rem(g, 2)]).wait()

    def fire_gather(g, slot):
        pltpu.async_copy(tab_hbm.at[vid.at[slot]], rows.at[slot],
                         gsem.at[slot])

    def drain_gather(g, slot):
        pltpu.make_async_copy(tab_hbm.at[vid.at[slot]], rows.at[slot],
                              gsem.at[slot]).wait()

    def fire_scatter(g, slot):
        pltpu.async_copy(rows.at[slot], accum.at[did.at[lax.rem(g, 3)]],
                         ssem, add=True)

    def drain_scatter(g, slot):
        pltpu.make_async_copy(rows.at[slot], accum.at[did.at[lax.rem(g, 3)]],
                              ssem).wait()

    fire_ids(0)
    fire_ids(1)
    zbuf = rows.at[1]

    def zfill(r, carry):
        def zfill_c(c, _):
            zbuf[r, pl.ds(c * 16, 16)] = jnp.zeros((16,), jnp.float32)
            return _
        return lax.fori_loop(0, D // 16, zfill_c, carry)
    lax.fori_loop(0, CH, zfill, 0)
    for k in range(SROWS // CH):
        pltpu.async_copy(zbuf, accum.at[pl.ds(sid_ * SROWS + k * CH, CH)],
                         ssem)
    drain_ids(0)
    fire_gather(0, 0)
    for k in range(SROWS // CH):
        pltpu.make_async_copy(zbuf, accum.at[pl.ds(sid_ * SROWS + k * CH, CH)],
                              ssem).wait()
    plsc.subcore_barrier()

    def piter(g, carry):
        slot = lax.rem(g, 2)

        @pl.when(g > 0)
        def _():
            drain_scatter(g - 1, 1 - slot)

        @pl.when(g + 1 < ng)
        def _():
            drain_ids(g + 1)
            fire_gather(g + 1, 1 - slot)

        drain_gather(g, slot)
        fire_scatter(g, slot)

        @pl.when(g + 2 < ng)
        def _():
            fire_ids(g + 2)
        return carry
    lax.fori_loop(0, ng, piter, 0)
    drain_scatter(ng - 1, lax.rem(ng - 1, 2))

    plsc.subcore_barrier()
    r0 = sid_ * SROWS
    pltpu.sync_copy(accum.at[pl.ds(r0, SROWS)],
                    out_hbm.at[pl.ds(cid * NEP + r0, SROWS)])


_agg = pl.kernel(
    _agg_body,
    out_type=jax.ShapeDtypeStruct((NC * NEP, D), jnp.float32),
    scratch_types=[
        pltpu.VMEM((2, CH), jnp.int32),
        pltpu.VMEM((3, CH), jnp.int32),
        pltpu.VMEM((2, CH, D), jnp.float32),
        pltpu.VMEM_SHARED((NEP, D), jnp.float32),
        pltpu.SemaphoreType.DMA((2,)),
        pltpu.SemaphoreType.DMA((2,)),
        pltpu.SemaphoreType.DMA,
    ],
    mesh=_mesh,
)


CW = 16
CWORDS = NEP * CW
CTILE = CWORDS // NS
CZ = 2048


def _counts_body(ids_hbm, out_hbm, dbuf, sbuf, ones, zbuf, accum, isem, ssem):
    cid = lax.axis_index("c")
    sid = lax.axis_index("s")
    wid = sid * NC + cid
    ng = NGBASE + jnp.where(wid < NGREM, 1, 0)
    wbase = (wid * NGBASE + jnp.minimum(wid, NGREM)) * CH

    def fire_ids(g):
        b = pl.multiple_of(wbase + g * CH, 8)
        pltpu.async_copy(ids_hbm.at[pl.ds(b, CH)], dbuf.at[lax.rem(g, 2)],
                         isem.at[lax.rem(g, 2)])

    def drain_ids(g):
        b = pl.multiple_of(wbase + g * CH, 8)
        pltpu.make_async_copy(ids_hbm.at[pl.ds(b, CH)],
                              dbuf.at[lax.rem(g, 2)],
                              isem.at[lax.rem(g, 2)]).wait()

    def fire_scatter(slot):
        pltpu.async_copy(ones, accum.at[sbuf.at[slot]], ssem, add=True)

    def drain_scatter(slot):
        pltpu.make_async_copy(ones, accum.at[sbuf.at[slot]], ssem).wait()

    fire_ids(0)
    fire_ids(1)

    def ofill(r, carry):
        ones[pl.ds(r * 16, 16)] = jnp.ones((16,), jnp.float32)
        return carry
    lax.fori_loop(0, CH // 16, ofill, 0)

    def zfill(r, carry):
        zbuf[pl.ds(r * 16, 16)] = jnp.zeros((16,), jnp.float32)
        return carry
    lax.fori_loop(0, CZ // 16, zfill, 0)
    for k in range(CTILE // CZ):
        pltpu.async_copy(zbuf, accum.at[pl.ds(sid * CTILE + k * CZ, CZ)],
                         ssem)
    for k in range(CTILE // CZ):
        pltpu.make_async_copy(zbuf, accum.at[pl.ds(sid * CTILE + k * CZ, CZ)],
                              ssem).wait()
    plsc.subcore_barrier()

    def body(g, carry):
        slot = lax.rem(g, 2)

        @pl.when(g > 0)
        def _():
            drain_scatter(1 - slot)

        drain_ids(g)

        def scale(j, carry2):
            sbuf[slot, pl.ds(j * 16, 16)] = dbuf[slot, pl.ds(j * 16, 16)] * CW
            return carry2
        lax.fori_loop(0, CH // 16, scale, 0)
        fire_scatter(slot)

        @pl.when(g + 2 < ng)
        def _():
            fire_ids(g + 2)
        return carry
    lax.fori_loop(0, ng, body, 0)
    drain_scatter(lax.rem(ng - 1, 2))

    plsc.subcore_barrier()
    r0 = sid * CTILE
    pltpu.sync_copy(accum.at[pl.ds(r0, CTILE)],
                    out_hbm.at[pl.ds(cid * CWORDS + r0, CTILE)])


_counts = pl.kernel(
    _counts_body,
    out_type=jax.ShapeDtypeStruct((NC * CWORDS,), jnp.float32),
    scratch_types=[
        pltpu.VMEM((2, CH), jnp.int32),
        pltpu.VMEM((2, CH), jnp.int32),
        pltpu.VMEM((CH,), jnp.float32),
        pltpu.VMEM((CZ,), jnp.float32),
        pltpu.VMEM_SHARED((CWORDS,), jnp.float32),
        pltpu.SemaphoreType.DMA((2,)),
        pltpu.SemaphoreType.DMA,
    ],
    mesh=_mesh,
)


BR = 1000


def _mm_body(x_ref, w_ref, o_ref):
    o_ref[...] = lax.dot_general(
        x_ref[...], w_ref[...], (((1,), (1,)), ((), ())),
        preferred_element_type=jnp.float32)


def _mm(x, w):
    return pl.pallas_call(
        _mm_body,
        grid=(N // BR,),
        in_specs=[
            pl.BlockSpec((BR, D), lambda i: (i, 0)),
            pl.BlockSpec((D, D), lambda i: (0, 0)),
        ],
        out_specs=pl.BlockSpec((BR, D), lambda i: (i, 0)),
        out_shape=jax.ShapeDtypeStruct((N, D), jnp.float32),
    )(x, w)


def _ec_body(p0_ref, p1_ref, c0_ref, c1_ref, o_ref):
    cnt = c0_ref[0][:, :1] + c1_ref[0][:, :1]
    rec = 1.0 / jnp.maximum(cnt, 1.0)
    o_ref[...] = (p0_ref[0] + p1_ref[0]) * rec


def _edge_combine(ep, cp):
    ep3 = ep.reshape(NC, NEP, D)
    cp3 = cp.reshape(NC, NEP, CW)
    return pl.pallas_call(
        _ec_body,
        grid=(NE // BR,),
        in_specs=[
            pl.BlockSpec((1, BR, D), lambda i: (0, i, 0)),
            pl.BlockSpec((1, BR, D), lambda i: (1, i, 0)),
            pl.BlockSpec((1, BR, CW), lambda i: (0, i, 0)),
            pl.BlockSpec((1, BR, CW), lambda i: (1, i, 0)),
        ],
        out_specs=pl.BlockSpec((BR, D), lambda i: (i, 0)),
        out_shape=jax.ShapeDtypeStruct((NE, D), jnp.float32),
    )(ep3, ep3, cp3, cp3)


def _vc1_body(eps_ref, y_ref, v0_ref, v1_ref, w_ref, o_ref):
    x = (1.0 + eps_ref[0]) * y_ref[...] + v0_ref[0] + v1_ref[0]
    x = jnp.maximum(x, 0.0)
    o_ref[...] = lax.dot_general(
        x, w_ref[...], (((1,), (1,)), ((), ())),
        preferred_element_type=jnp.float32)


def _vert_combine_relu_mm(y, vp, w2, eps):
    vp3 = vp.reshape(NC, NEP, D)
    return pl.pallas_call(
        _vc1_body,
        grid=(N // BR,),
        in_specs=[
            pl.BlockSpec(memory_space=pltpu.SMEM),
            pl.BlockSpec((BR, D), lambda i: (i, 0)),
            pl.BlockSpec((1, BR, D), lambda i: (0, i, 0)),
            pl.BlockSpec((1, BR, D), lambda i: (1, i, 0)),
            pl.BlockSpec((D, D), lambda i: (0, 0)),
        ],
        out_specs=pl.BlockSpec((BR, D), lambda i: (i, 0)),
        out_shape=jax.ShapeDtypeStruct((N, D), jnp.float32),
    )(eps, y, vp3, vp3, w2)


def _vc2_body(eps_ref, y_ref, v0_ref, v1_ref, o_ref):
    o_ref[...] = (1.0 + eps_ref[0]) * y_ref[...] + v0_ref[0] + v1_ref[0]


def _vert_combine(y, vp, eps):
    vp3 = vp.reshape(NC, NEP, D)
    return pl.pallas_call(
        _vc2_body,
        grid=(N // BR,),
        in_specs=[
            pl.BlockSpec(memory_space=pltpu.SMEM),
            pl.BlockSpec((BR, D), lambda i: (i, 0)),
            pl.BlockSpec((1, BR, D), lambda i: (0, i, 0)),
            pl.BlockSpec((1, BR, D), lambda i: (1, i, 0)),
        ],
        out_specs=pl.BlockSpec((BR, D), lambda i: (i, 0)),
        out_shape=jax.ShapeDtypeStruct((N, D), jnp.float32),
    )(eps, y, vp3, vp3)


def kernel(x, hyperedge_index, W1, eps1, W2, eps2):
    vertex = hyperedge_index[0]
    edges = hyperedge_index[1]

    cp = _counts(edges)

    y1 = _mm(x, W1)
    ep1 = _agg(y1, vertex, edges)
    xe1 = _edge_combine(ep1, cp)
    vp1 = _agg(xe1, edges, vertex)
    y2 = _vert_combine_relu_mm(y1, vp1, W2, eps1)

    ep2 = _agg(y2, vertex, edges)
    xe2 = _edge_combine(ep2, cp)
    vp2 = _agg(xe2, edges, vertex)
    x2 = _vert_combine(y2, vp2, eps2)

    return (x2, xe2)
